# baseline (speedup 1.0000x reference)
import jax
import jax.numpy as jnp
from jax.experimental import pallas as pl
from jax.experimental.pallas import tpu as pltpu

RADIUS = 0.2
C_IN = 32
C_OUT = 32
HIDDEN = 64


def _conv_kernel(gaT_ref, gb_ref, w1_ref, w1T_ref, fb_ref, w2t_ref,
                 out_ref, hm_ref, gs_ref):
    n = gaT_ref.shape[2]
    gaT = gaT_ref[0]
    gb = gb_ref[0]
    gblk = jnp.dot(fb_ref[0].astype(jnp.bfloat16), w2t_ref[...].astype(jnp.bfloat16),
                   preferred_element_type=jnp.float32)
    for k in range(HIDDEN):
        gs_ref[k * n:(k + 1) * n, :] = (
            gblk[:, k * C_OUT:(k + 1) * C_OUT].astype(jnp.bfloat16))
    paT = jnp.dot(w1T_ref[...], gaT, preferred_element_type=jnp.float32,
                  precision=jax.lax.Precision.HIGHEST)
    pb = jnp.dot(gb, w1_ref[...], preferred_element_type=jnp.float32,
                 precision=jax.lax.Precision.HIGHEST)
    d0 = gaT[0:1, :] - gb[:, 0:1]
    d1 = gaT[1:2, :] - gb[:, 1:2]
    d2 = gaT[2:3, :] - gb[:, 2:3]
    n2 = d0 * d0 + d1 * d1 + d2 * d2
    inmask = jnp.sqrt(n2) < RADIUS
    mb = inmask.astype(jnp.bfloat16)
    zero_b = jnp.zeros((), jnp.bfloat16)
    for k in range(HIDDEN):
        hk = (pb[:, k:k + 1] - paT[k:k + 1, :]).astype(jnp.bfloat16)
        hm_ref[k * n:(k + 1) * n, :] = jnp.maximum(hk, zero_b) * mb
    outT = jax.lax.dot_general(
        gs_ref[...], hm_ref[...],
        dimension_numbers=(((0,), (0,)), ((), ())),
        preferred_element_type=jnp.float32,
    )
    out_ref[0] = outT.T


def kernel(features, geometry, W1, W2):
    batch, n, _ = geometry.shape

    w2t = (W2.reshape(HIDDEN, C_OUT, C_IN).transpose(2, 0, 1)
           .reshape(C_IN, HIDDEN * C_OUT))
    gT = geometry.transpose(0, 2, 1)
    w1T = W1.T

    out = pl.pallas_call(
        _conv_kernel,
        grid=(batch,),
        in_specs=[
            pl.BlockSpec((1, 3, n), lambda z: (z, 0, 0)),
            pl.BlockSpec((1, n, 3), lambda z: (z, 0, 0)),
            pl.BlockSpec((3, HIDDEN), lambda z: (0, 0)),
            pl.BlockSpec((HIDDEN, 3), lambda z: (0, 0)),
            pl.BlockSpec((1, n, C_IN), lambda z: (z, 0, 0)),
            pl.BlockSpec((C_IN, HIDDEN * C_OUT), lambda z: (0, 0)),
        ],
        out_specs=pl.BlockSpec((1, n, C_OUT), lambda z: (z, 0, 0)),
        out_shape=jax.ShapeDtypeStruct((batch, n, C_OUT), jnp.float32),
        scratch_shapes=[
            pltpu.VMEM((HIDDEN * n, n), jnp.bfloat16),
            pltpu.VMEM((HIDDEN * n, C_OUT), jnp.bfloat16),
        ],
    )(gT, geometry, W1, w1T, features, w2t)
    return out


# speedup vs baseline: 1.0342x; 1.0342x over previous
import jax
import jax.numpy as jnp
from jax.experimental import pallas as pl
from jax.experimental.pallas import tpu as pltpu

RADIUS = 0.2
C_IN = 32
C_OUT = 32
HIDDEN = 64


def _conv_kernel(gb_ref, w1_ref, fb_ref, w2t_ref,
                 out_ref, hm_ref, gs_ref):
    n = gb_ref.shape[1]
    gb = gb_ref[0]
    gaT = gb.T
    gblk = jnp.dot(fb_ref[0].astype(jnp.bfloat16), w2t_ref[...].astype(jnp.bfloat16),
                   preferred_element_type=jnp.float32)
    for k in range(HIDDEN):
        gs_ref[k * n:(k + 1) * n, :] = (
            gblk[:, k * C_OUT:(k + 1) * C_OUT].astype(jnp.bfloat16))
    pb = jnp.dot(gb, w1_ref[...], preferred_element_type=jnp.float32,
                 precision=jax.lax.Precision.HIGHEST)
    paT = pb.T
    d0 = gaT[0:1, :] - gb[:, 0:1]
    d1 = gaT[1:2, :] - gb[:, 1:2]
    d2 = gaT[2:3, :] - gb[:, 2:3]
    n2 = d0 * d0 + d1 * d1 + d2 * d2
    inmask = jnp.sqrt(n2) < RADIUS
    mb = inmask.astype(jnp.bfloat16)
    zero_b = jnp.zeros((), jnp.bfloat16)
    for k in range(HIDDEN):
        hk = (pb[:, k:k + 1] - paT[k:k + 1, :]).astype(jnp.bfloat16)
        hm_ref[k * n:(k + 1) * n, :] = jnp.maximum(hk, zero_b) * mb
    outT = jax.lax.dot_general(
        gs_ref[...], hm_ref[...],
        dimension_numbers=(((0,), (0,)), ((), ())),
        preferred_element_type=jnp.float32,
    )
    out_ref[0] = outT.T


def kernel(features, geometry, W1, W2):
    batch, n, _ = geometry.shape

    w2t = (W2.reshape(HIDDEN, C_OUT, C_IN).transpose(2, 0, 1)
           .reshape(C_IN, HIDDEN * C_OUT))

    out = pl.pallas_call(
        _conv_kernel,
        grid=(batch,),
        in_specs=[
            pl.BlockSpec((1, n, 3), lambda z: (z, 0, 0)),
            pl.BlockSpec((3, HIDDEN), lambda z: (0, 0)),
            pl.BlockSpec((1, n, C_IN), lambda z: (z, 0, 0)),
            pl.BlockSpec((C_IN, HIDDEN * C_OUT), lambda z: (0, 0)),
        ],
        out_specs=pl.BlockSpec((1, n, C_OUT), lambda z: (z, 0, 0)),
        out_shape=jax.ShapeDtypeStruct((batch, n, C_OUT), jnp.float32),
        scratch_shapes=[
            pltpu.VMEM((HIDDEN * n, n), jnp.bfloat16),
            pltpu.VMEM((HIDDEN * n, C_OUT), jnp.bfloat16),
        ],
    )(geometry, W1, features, w2t)
    return out
